# baseline (speedup 1.0000x reference)
import numpy as np
import jax
import jax.numpy as jnp
from jax import lax
from jax.experimental import pallas as pl
from jax.experimental.pallas import tpu as pltpu
from jax.experimental.pallas import tpu_sc as plsc

_B, _N, _D = 4, 8193, 1024
_G = 2048
_NC, _NS, _L = 2, 16, 16
_NW = _NC * _NS
_ROWS = _B * _G
_RPW = _ROWS // _NW
_C = 8
_E = 5 * _C
_NCHUNK = _RPW // _C
_TPW = _RPW * 5


_g = np.arange(_G, dtype=np.int32)
_nodes = np.stack([_g, 4 * _g + 1, 4 * _g + 2, 4 * _g + 3, 4 * _g + 4],
                  axis=1)
_NODE_TAB = (np.arange(_B, dtype=np.int32)[:, None, None] * _N
             + _nodes[None]).reshape(-1)


def _body(h_hbm, nt_hbm, out_hbm, stage, obuf, idxtab, sem_g, sem_o):
    cid = lax.axis_index("c")
    sid = lax.axis_index("s")
    wid = sid * _NC + cid
    base = wid * _RPW

    pltpu.sync_copy(nt_hbm.at[pl.ds(wid * _TPW, _TPW)], idxtab)

    def issue_gather(s, i):
        pltpu.async_copy(
            h_hbm.at[idxtab.at[pl.ds(i * _E, _E)]], stage[s], sem_g[s])

    def wait_gather(s, i):
        pltpu.make_async_copy(
            h_hbm.at[idxtab.at[pl.ds(i * _E, _E)]], stage[s],
            sem_g[s]).wait()

    def issue_scatter(s, i):
        pltpu.async_copy(
            obuf[s], out_hbm.at[pl.ds(base + i * _C, _C)], sem_o[s])

    def wait_scatter(s, i):
        pltpu.make_async_copy(
            obuf[s], out_hbm.at[pl.ds(base + i * _C, _C)], sem_o[s]).wait()

    def compute(s):
        st = stage[s]
        ob = obuf[s]

        def row(c, carry):
            r = 5 * c
            for k in range(_D // _L):
                sl = pl.ds(k * _L, _L)
                v = st[r, sl] + st[r + 1, sl]
                v = v + st[r + 2, sl]
                v = v + st[r + 3, sl]
                v = v + st[r + 4, sl]
                ob[c, sl] = v * 0.2
            return carry

        lax.fori_loop(0, _C, row, 0)

    for s in range(2):
        issue_gather(s, jnp.int32(s))

    def step(t, carry):
        i0 = 2 * t
        for s in range(2):
            i = i0 + s
            wait_gather(s, i)

            @pl.when(i0 >= 2)
            def _():
                wait_scatter(s, i - 2)

            compute(s)
            issue_scatter(s, i)

            @pl.when(i0 + 2 < _NCHUNK)
            def _():
                issue_gather(s, i + 2)

        return carry

    lax.fori_loop(0, _NCHUNK // 2, step, 0)
    for s in range(2):
        wait_scatter(s, jnp.int32(_NCHUNK - 2 + s))


@jax.jit
def _run(h):
    h2 = h.reshape(_B * _N, _D)
    call = pl.kernel(
        _body,
        out_type=jax.ShapeDtypeStruct((_ROWS, _D), jnp.float32),
        mesh=plsc.VectorSubcoreMesh(
            core_axis_name="c", subcore_axis_name="s",
            num_cores=_NC, num_subcores=_NS),
        scratch_types=[
            [pltpu.VMEM((_E, _D), jnp.float32) for _ in range(2)],
            [pltpu.VMEM((_C, _D), jnp.float32) for _ in range(2)],
            pltpu.VMEM((_TPW,), jnp.int32),
            [pltpu.SemaphoreType.DMA for _ in range(2)],
            [pltpu.SemaphoreType.DMA for _ in range(2)],
        ],
    )
    out2 = call(h2, jnp.asarray(_NODE_TAB))
    return out2.reshape(_B, _G, _D)


def kernel(h):
    return _run(h)


# speedup vs baseline: 1.3091x; 1.3091x over previous
import numpy as np
import jax
import jax.numpy as jnp
from jax import lax
from jax.experimental import pallas as pl
from jax.experimental.pallas import tpu as pltpu
from jax.experimental.pallas import tpu_sc as plsc

_B, _N, _D = 4, 8193, 1024
_G = 2048
_NC, _NS, _L = 2, 16, 16
_NW = _NC * _NS
_ROWS = _B * _G
_RPW = _ROWS // _NW
_C = 8
_E = 5 * _C
_NCHUNK = _RPW // _C
_TPW = _RPW * 5


_g = np.arange(_G, dtype=np.int32)
_nodes = np.stack([_g, 4 * _g + 1, 4 * _g + 2, 4 * _g + 3, 4 * _g + 4],
                  axis=1)
_NODE_TAB = (np.arange(_B, dtype=np.int32)[:, None, None] * _N
             + _nodes[None]).reshape(-1)


def _body(h_hbm, nt_hbm, out_hbm, stage, obuf, idxtab, sem_g, sem_o):
    cid = lax.axis_index("c")
    sid = lax.axis_index("s")
    wid = sid * _NC + cid
    base = wid * _RPW

    pltpu.sync_copy(nt_hbm.at[pl.ds(wid * _TPW, _TPW)], idxtab)

    def issue_gather(s, i):
        pltpu.async_copy(
            h_hbm.at[idxtab.at[pl.ds(i * _E, _E)]], stage[s], sem_g[s])

    def wait_gather(s, i):
        pltpu.make_async_copy(
            h_hbm.at[idxtab.at[pl.ds(i * _E, _E)]], stage[s],
            sem_g[s]).wait()

    def issue_scatter(s, i):
        pltpu.async_copy(
            obuf[s], out_hbm.at[pl.ds(base + i * _C, _C)], sem_o[s])

    def wait_scatter(s, i):
        pltpu.make_async_copy(
            obuf[s], out_hbm.at[pl.ds(base + i * _C, _C)], sem_o[s]).wait()

    def compute(s):
        st = stage[s]
        ob = obuf[s]

        def row(c, carry):
            r = 5 * c
            for k4 in range(0, _D // _L, 4):
                loads = [[st[r + j, pl.ds((k4 + u) * _L, _L)]
                          for j in range(5)] for u in range(4)]
                for u in range(4):
                    l = loads[u]
                    v = (l[0] + l[1]) + (l[2] + l[3])
                    ob[c, pl.ds((k4 + u) * _L, _L)] = (v + l[4]) * 0.2
            return carry

        lax.fori_loop(0, _C, row, 0)

    for s in range(2):
        issue_gather(s, jnp.int32(s))

    def step(t, carry):
        i0 = 2 * t
        for s in range(2):
            i = i0 + s
            wait_gather(s, i)

            @pl.when(i0 >= 2)
            def _():
                wait_scatter(s, i - 2)

            compute(s)
            issue_scatter(s, i)

            @pl.when(i0 + 2 < _NCHUNK)
            def _():
                issue_gather(s, i + 2)

        return carry

    lax.fori_loop(0, _NCHUNK // 2, step, 0)
    for s in range(2):
        wait_scatter(s, jnp.int32(_NCHUNK - 2 + s))


@jax.jit
def _run(h):
    h2 = h.reshape(_B * _N, _D)
    call = pl.kernel(
        _body,
        out_type=jax.ShapeDtypeStruct((_ROWS, _D), jnp.float32),
        mesh=plsc.VectorSubcoreMesh(
            core_axis_name="c", subcore_axis_name="s",
            num_cores=_NC, num_subcores=_NS),
        scratch_types=[
            [pltpu.VMEM((_E, _D), jnp.float32) for _ in range(2)],
            [pltpu.VMEM((_C, _D), jnp.float32) for _ in range(2)],
            pltpu.VMEM((_TPW,), jnp.int32),
            [pltpu.SemaphoreType.DMA for _ in range(2)],
            [pltpu.SemaphoreType.DMA for _ in range(2)],
        ],
    )
    out2 = call(h2, jnp.asarray(_NODE_TAB))
    return out2.reshape(_B, _G, _D)


def kernel(h):
    return _run(h)
